# baseline (speedup 1.0000x reference)
import functools
import math

import jax
import jax.numpy as jnp
from jax import lax
from jax.experimental import pallas as pl
from jax.experimental.pallas import tpu as pltpu
from jax.experimental.pallas import tpu_sc as plsc

HID = 1024
HEAD_DIM = 64
BASE = 10000.0
EPS = 1e-12

NC = 2
NS = 16
NW = NC * NS

TOK = 8192
TPW = TOK // NW
CH = 32
NCH = TPW // CH

_sc_mesh = plsc.VectorSubcoreMesh(
    core_axis_name="c", subcore_axis_name="s", num_cores=NC, num_subcores=NS
)


def _make_gather_sc(n_tok):
    tpw = n_tok // NW
    nch = tpw // CH

    nbuf = 3

    @functools.partial(
        pl.kernel,
        out_type=jax.ShapeDtypeStruct((n_tok, HID), jnp.float32),
        mesh=_sc_mesh,
        scratch_types=[
            pltpu.VMEM((nch, CH), jnp.int32),
            *([pltpu.VMEM((CH, HID), jnp.float32)] * nbuf),
            *([pltpu.SemaphoreType.DMA] * nbuf),
            *([pltpu.SemaphoreType.DMA] * nbuf),
        ],
    )
    def _gather_sc(ids_hbm, table_hbm, out_hbm, idx_v, *rest):
        bufs = rest[:nbuf]
        gsems = rest[nbuf:2 * nbuf]
        osems = rest[2 * nbuf:3 * nbuf]
        wid = lax.axis_index("s") * NC + lax.axis_index("c")
        pltpu.sync_copy(ids_hbm.at[wid], idx_v)

        def gather(c):
            return pltpu.make_async_copy(
                table_hbm.at[idx_v.at[c]], bufs[c % nbuf], gsems[c % nbuf]
            )

        def out(c):
            return pltpu.make_async_copy(
                bufs[c % nbuf],
                out_hbm.at[pl.ds(wid * tpw + c * CH, CH)],
                osems[c % nbuf],
            )

        gather(0).start()
        if nch > 1:
            gather(1).start()
        for c in range(nch):
            gather(c).wait()
            out(c).start()
            n = c + 2
            if n < nch:
                if n >= nbuf:
                    out(n - nbuf).wait()
                gather(n).start()
        for c in range(max(0, nch - nbuf), nch):
            out(c).wait()

    return _gather_sc


def _ln_math(x, g, b):
    mu = jnp.mean(x, axis=1, keepdims=True)
    xc = x - mu
    var = jnp.mean(xc * xc, axis=1, keepdims=True)
    return xc * lax.rsqrt(var + EPS) * g + b


def _ln_body(rows_ref, tt_ref, g_ref, b_ref, out_ref):
    out_ref[...] = _ln_math(rows_ref[...] + tt_ref[...], g_ref[...], b_ref[...])


def _ln_body_acc(rows_ref, tt_ref, g_ref, b_ref, prev_ref, out_ref):
    del prev_ref
    out_ref[...] = _ln_math(rows_ref[...] + tt_ref[...], g_ref[...], b_ref[...])


TB = 2048


def _ln_tc_seg(rows, tt0, gamma, beta, seg, prev):
    n_tok = rows.shape[0]
    nblk = n_tok // TB
    off = seg * nblk
    row_spec = pl.BlockSpec((TB, HID), lambda i: (i, 0))
    chan_spec = pl.BlockSpec((1, HID), lambda i: (0, 0))
    out_spec = pl.BlockSpec((TB, HID), lambda i, o=off: (o + i, 0))
    args = [rows, tt0.reshape(1, HID), gamma.reshape(1, HID), beta.reshape(1, HID)]
    in_specs = [row_spec, chan_spec, chan_spec, chan_spec]
    body = _ln_body
    kwargs = {}
    if seg > 0:
        args.append(prev)
        in_specs.append(pl.BlockSpec(memory_space=pl.ANY))
        body = _ln_body_acc
        kwargs["input_output_aliases"] = {4: 0}
    return pl.pallas_call(
        body,
        grid=(nblk,),
        in_specs=in_specs,
        out_specs=out_spec,
        out_shape=jax.ShapeDtypeStruct((TOK, HID), jnp.float32),
        **kwargs,
    )(*args)


def _rope_body(cos_ref, sin_ref):
    s_len, d = cos_ref.shape
    half = d // 2
    pos = lax.broadcasted_iota(jnp.int32, (s_len, half), 0).astype(jnp.float32)
    i = lax.broadcasted_iota(jnp.int32, (s_len, half), 1).astype(jnp.float32)
    inv_freq = jnp.exp(i * (-2.0 * math.log(BASE) / d))
    ang = pos * inv_freq
    c = jnp.cos(ang)
    s = jnp.sin(ang)
    cos_ref[:, :half] = c
    cos_ref[:, half:] = c
    sin_ref[:, :half] = s
    sin_ref[:, half:] = s


def _rope_tc(b, s):
    cos_c, sin_c = pl.pallas_call(
        _rope_body,
        out_shape=(
            jax.ShapeDtypeStruct((s, HEAD_DIM), jnp.float32),
            jax.ShapeDtypeStruct((s, HEAD_DIM), jnp.float32),
        ),
    )()
    rope_cos = jnp.broadcast_to(cos_c[None, :, None, :], (b, s, 1, HEAD_DIM))
    rope_sin = jnp.broadcast_to(sin_c[None, :, None, :], (b, s, 1, HEAD_DIM))
    return rope_cos, rope_sin


L = 16
FCH = 16
FNCH = TPW // FCH
NJ = HID // L
_RSQRT_MAGIC = 0x5F3759DF


def _lane_sum(x):
    dnums = lax.GatherDimensionNumbers(
        offset_dims=(), collapsed_slice_dims=(0,), start_index_map=(0,)
    )
    lane = lax.iota(jnp.int32, L)
    for sh in (8, 4, 2, 1):
        perm = jnp.reshape((lane + sh) & (L - 1), (L, 1))
        x = x + lax.gather(
            x, perm, dnums, (1,), mode=lax.GatherScatterMode.PROMISE_IN_BOUNDS
        )
    return x


def _rsqrt_newton(va):
    bits = lax.bitcast_convert_type(va, jnp.int32)
    y = lax.bitcast_convert_type(_RSQRT_MAGIC - (bits >> 1), jnp.float32)
    for _ in range(3):
        y = y * (1.5 - 0.5 * va * y * y)
    return y


@functools.partial(
    pl.kernel,
    out_type=jax.ShapeDtypeStruct((TOK, HID), jnp.float32),
    mesh=_sc_mesh,
    scratch_types=[
        pltpu.VMEM((FNCH, FCH), jnp.int32),
        pltpu.VMEM((FCH, HID), jnp.float32),
        pltpu.VMEM((FCH, HID), jnp.float32),
        pltpu.VMEM((FCH, HID), jnp.float32),
        pltpu.VMEM((FCH, HID), jnp.float32),
        pltpu.VMEM((HID,), jnp.float32),
        pltpu.VMEM((HID,), jnp.float32),
        pltpu.VMEM((HID,), jnp.float32),
        pltpu.SemaphoreType.DMA,
        pltpu.SemaphoreType.DMA,
        pltpu.SemaphoreType.DMA,
        pltpu.SemaphoreType.DMA,
    ],
)
def _emb_ln_fused_sc(ids_hbm, table_hbm, tt_hbm, g_hbm, b_hbm, out_hbm,
                     idx_v, buf_a, buf_b, ob_a, ob_b, tt_v, g_v, b_v,
                     gs_a, gs_b, os_a, os_b):
    wid = lax.axis_index("s") * NC + lax.axis_index("c")
    pltpu.sync_copy(ids_hbm.at[wid], idx_v)
    pltpu.sync_copy(tt_hbm, tt_v)
    pltpu.sync_copy(g_hbm, g_v)
    pltpu.sync_copy(b_hbm, b_v)
    bufs = (buf_a, buf_b)
    obufs = (ob_a, ob_b)
    gsems = (gs_a, gs_b)
    osems = (os_a, os_b)

    def gather(c, b):
        return pltpu.make_async_copy(table_hbm.at[idx_v.at[c]], bufs[b], gsems[b])

    def outcp(c, b):
        return pltpu.make_async_copy(
            obufs[b], out_hbm.at[pl.ds(wid * TPW + c * FCH, FCH)], osems[b]
        )

    gather(0, 0).start()
    gather(1, 1).start()

    def process(buf, obuf, r, carry):
        del carry
        accs = [jnp.zeros((L,), jnp.float32) for _ in range(4)]
        acc2s = [jnp.zeros((L,), jnp.float32) for _ in range(4)]
        for j in range(NJ):
            sl = pl.ds(j * L, L)
            x = buf[r, sl] + tt_v[sl]
            accs[j % 4] = accs[j % 4] + x
            acc2s[j % 4] = acc2s[j % 4] + x * x
        s1 = _lane_sum((accs[0] + accs[1]) + (accs[2] + accs[3]))
        s2 = _lane_sum((acc2s[0] + acc2s[1]) + (acc2s[2] + acc2s[3]))
        muv = s1 * (1.0 / HID)
        varv = s2 * (1.0 / HID) - muv * muv
        inv = _rsqrt_newton(varv + EPS)
        for j in range(NJ):
            sl = pl.ds(j * L, L)
            x = buf[r, sl] + tt_v[sl]
            obuf[r, sl] = (x - muv) * inv * g_v[sl] + b_v[sl]
        return 0

    def half(i, b):
        c = 2 * i + b
        gather(c, b).wait()
        lax.fori_loop(0, FCH, functools.partial(process, bufs[b], obufs[b]), 0)

        @pl.when(i > 0)
        def _drain_prev_out():
            outcp(c - 2, b).wait()

        outcp(c, b).start()

        @pl.when(c + 2 < FNCH)
        def _next_gather():
            gather(c + 2, b).start()

    def chunk_pair(i, carry):
        half(i, 0)
        half(i, 1)
        return carry

    lax.fori_loop(0, FNCH // 2, chunk_pair, 0)
    outcp(FNCH - 2, 0).wait()
    outcp(FNCH - 1, 1).wait()


def kernel(input_ids, word_emb, token_type_emb, ln_gamma, ln_beta):
    b, s = input_ids.shape
    ids = input_ids.reshape(NW, FNCH, FCH).astype(jnp.int32)
    tt0 = token_type_emb[0]

    emb_flat = _emb_ln_fused_sc(ids, word_emb, tt0, ln_gamma, ln_beta)
    embeddings = emb_flat.reshape(b, s, HID)

    rope_cos, rope_sin = _rope_tc(b, s)

    attention_mask = jnp.ones((b, s), dtype=jnp.float32)
    return embeddings, attention_mask, rope_cos, rope_sin


# speedup vs baseline: 2.8538x; 2.8538x over previous
import functools
import math

import jax
import jax.numpy as jnp
from jax import lax
from jax.experimental import pallas as pl
from jax.experimental.pallas import tpu as pltpu
from jax.experimental.pallas import tpu_sc as plsc

HID = 1024
HEAD_DIM = 64
BASE = 10000.0
EPS = 1e-12

NC = 2
NS = 16
NW = NC * NS

TOK = 8192
TPW = TOK // NW
CH = 32
NCH = TPW // CH

_sc_mesh = plsc.VectorSubcoreMesh(
    core_axis_name="c", subcore_axis_name="s", num_cores=NC, num_subcores=NS
)


def _make_gather_sc(n_tok):
    tpw = n_tok // NW
    nch = tpw // CH

    nbuf = 3

    @functools.partial(
        pl.kernel,
        out_type=jax.ShapeDtypeStruct((n_tok, HID), jnp.float32),
        mesh=_sc_mesh,
        scratch_types=[
            pltpu.VMEM((nch, CH), jnp.int32),
            *([pltpu.VMEM((CH, HID), jnp.float32)] * nbuf),
            *([pltpu.SemaphoreType.DMA] * nbuf),
            *([pltpu.SemaphoreType.DMA] * nbuf),
        ],
    )
    def _gather_sc(ids_hbm, table_hbm, out_hbm, idx_v, *rest):
        bufs = rest[:nbuf]
        gsems = rest[nbuf:2 * nbuf]
        osems = rest[2 * nbuf:3 * nbuf]
        wid = lax.axis_index("s") * NC + lax.axis_index("c")
        pltpu.sync_copy(ids_hbm.at[wid], idx_v)

        def gather(c):
            return pltpu.make_async_copy(
                table_hbm.at[idx_v.at[c]], bufs[c % nbuf], gsems[c % nbuf]
            )

        def out(c):
            return pltpu.make_async_copy(
                bufs[c % nbuf],
                out_hbm.at[pl.ds(wid * tpw + c * CH, CH)],
                osems[c % nbuf],
            )

        gather(0).start()
        if nch > 1:
            gather(1).start()
        for c in range(nch):
            gather(c).wait()
            out(c).start()
            n = c + 2
            if n < nch:
                if n >= nbuf:
                    out(n - nbuf).wait()
                gather(n).start()
        for c in range(max(0, nch - nbuf), nch):
            out(c).wait()

    return _gather_sc


def _ln_math(x, g, b):
    mu = jnp.mean(x, axis=1, keepdims=True)
    xc = x - mu
    var = jnp.mean(xc * xc, axis=1, keepdims=True)
    return xc * lax.rsqrt(var + EPS) * g + b


def _ln_body(rows_ref, tt_ref, g_ref, b_ref, out_ref):
    out_ref[...] = _ln_math(rows_ref[...] + tt_ref[...], g_ref[...], b_ref[...])


def _ln_body_acc(rows_ref, tt_ref, g_ref, b_ref, prev_ref, out_ref):
    del prev_ref
    out_ref[...] = _ln_math(rows_ref[...] + tt_ref[...], g_ref[...], b_ref[...])


TB = 2048


def _ln_tc_seg(rows, tt0, gamma, beta, seg, prev):
    n_tok = rows.shape[0]
    nblk = n_tok // TB
    off = seg * nblk
    row_spec = pl.BlockSpec((TB, HID), lambda i: (i, 0))
    chan_spec = pl.BlockSpec((1, HID), lambda i: (0, 0))
    out_spec = pl.BlockSpec((TB, HID), lambda i, o=off: (o + i, 0))
    args = [rows, tt0.reshape(1, HID), gamma.reshape(1, HID), beta.reshape(1, HID)]
    in_specs = [row_spec, chan_spec, chan_spec, chan_spec]
    body = _ln_body
    kwargs = {}
    if seg > 0:
        args.append(prev)
        in_specs.append(pl.BlockSpec(memory_space=pl.ANY))
        body = _ln_body_acc
        kwargs["input_output_aliases"] = {4: 0}
    return pl.pallas_call(
        body,
        grid=(nblk,),
        in_specs=in_specs,
        out_specs=out_spec,
        out_shape=jax.ShapeDtypeStruct((TOK, HID), jnp.float32),
        **kwargs,
    )(*args)


def _rope_body(cos_ref, sin_ref):
    s_len, d = cos_ref.shape
    half = d // 2
    pos = lax.broadcasted_iota(jnp.int32, (s_len, half), 0).astype(jnp.float32)
    i = lax.broadcasted_iota(jnp.int32, (s_len, half), 1).astype(jnp.float32)
    inv_freq = jnp.exp(i * (-2.0 * math.log(BASE) / d))
    ang = pos * inv_freq
    c = jnp.cos(ang)
    s = jnp.sin(ang)
    cos_ref[:, :half] = c
    cos_ref[:, half:] = c
    sin_ref[:, :half] = s
    sin_ref[:, half:] = s


def _rope_tc(b, s):
    cos_c, sin_c = pl.pallas_call(
        _rope_body,
        out_shape=(
            jax.ShapeDtypeStruct((s, HEAD_DIM), jnp.float32),
            jax.ShapeDtypeStruct((s, HEAD_DIM), jnp.float32),
        ),
    )()
    rope_cos = jnp.broadcast_to(cos_c[None, :, None, :], (b, s, 1, HEAD_DIM))
    rope_sin = jnp.broadcast_to(sin_c[None, :, None, :], (b, s, 1, HEAD_DIM))
    return rope_cos, rope_sin


NSEG = 1
SEG = TOK // NSEG
_gather_seg = _make_gather_sc(SEG)


def kernel(input_ids, word_emb, token_type_emb, ln_gamma, ln_beta):
    b, s = input_ids.shape
    ids = input_ids.reshape(NSEG, NW, SEG // NW // CH, CH).astype(jnp.int32)
    tt0 = token_type_emb[0]

    rows = [_gather_seg(ids[k], word_emb) for k in range(NSEG)]
    emb_flat = None
    for k in range(NSEG):
        emb_flat = _ln_tc_seg(rows[k], tt0, ln_gamma, ln_beta, k, emb_flat)
    embeddings = emb_flat.reshape(b, s, HID)

    rope_cos, rope_sin = _rope_tc(b, s)

    attention_mask = jnp.ones((b, s), dtype=jnp.float32)
    return embeddings, attention_mask, rope_cos, rope_sin
